# baseline (speedup 1.0000x reference)
import functools

import jax
import jax.numpy as jnp
from jax import lax
from jax.experimental import pallas as pl
from jax.experimental.pallas import tpu as pltpu
from jax.experimental.pallas import tpu_sc as plsc

B = 4096
NT = 26
V = 100000
D = 64

NC = 2
NS = 16
NW = NC * NS
BPW = B // NW
SLOTS = 32


def _sc_gather_kernel(idx_hbm, table_hbm, out_hbm, idx_v, buf0, buf1, sem0, sem1):
    wid = lax.axis_index("s") * NC + lax.axis_index("c")
    b0 = wid * BPW

    pltpu.sync_copy(idx_hbm.at[:, pl.ds(b0, BPW)], idx_v)

    def add_base(t, carry):
        off = t * V
        for l in range(BPW // 16):
            sl = pl.ds(l * 16, 16)
            idx_v[t, sl] = idx_v[t, sl] + off
        return carry

    lax.fori_loop(0, NT, add_base, 0)

    bufs = (buf0, buf1)
    sems = (sem0, sem1)
    handles = [None, None]
    for t in range(NT):
        k = t % 2
        if handles[k] is not None:
            handles[k].wait()
            pltpu.sync_copy(bufs[k], out_hbm.at[pl.ds(b0, BPW), t - 2 + 1, :])
        handles[k] = pltpu.async_copy(table_hbm.at[idx_v.at[t]], bufs[k], sems[k])
    for t in (NT - 2, NT - 1):
        k = t % 2
        handles[k].wait()
        pltpu.sync_copy(bufs[k], out_hbm.at[pl.ds(b0, BPW), t + 1, :])


@functools.lru_cache(maxsize=None)
def _sc_gather():
    return pl.kernel(
        _sc_gather_kernel,
        mesh=plsc.VectorSubcoreMesh(core_axis_name="c", subcore_axis_name="s"),
        compiler_params=pltpu.CompilerParams(use_tc_tiling_on_sc=False),
        out_type=jax.ShapeDtypeStruct((B, SLOTS, D), jnp.float32),
        scratch_types=[
            pltpu.VMEM((NT, BPW), jnp.int32),
            pltpu.VMEM((BPW, D), jnp.float32),
            pltpu.VMEM((BPW, D), jnp.float32),
            pltpu.SemaphoreType.DMA,
            pltpu.SemaphoreType.DMA,
        ],
    )


def _tc_body(dense_ref, ly_ref, bw0, bb0, bw1, bb1, bw2, bb2,
             tw0, tb0, tw1, tb1, tw2, tb2, out_ref):
    f32 = jnp.float32
    x = dense_ref[...]
    h = jnp.maximum(jnp.dot(x, bw0[...], preferred_element_type=f32) + bb0[...], 0.0)
    h = jnp.maximum(jnp.dot(h, bw1[...], preferred_element_type=f32) + bb1[...], 0.0)
    xb = jnp.maximum(jnp.dot(h, bw2[...], preferred_element_type=f32) + bb2[...], 0.0)

    bb = xb.shape[0]
    ly = ly_ref[...]
    slot = lax.broadcasted_iota(jnp.int32, (bb, SLOTS, D), 1)
    tall = jnp.where(slot == 0, xb[:, None, :],
                     jnp.where(slot <= NT, ly, 0.0))

    pieces = [xb]
    for i in range(1, NT + 1):
        zi = jnp.sum(tall * tall[:, i:i + 1, :], axis=2)
        pieces.append(zi)
    r = jnp.concatenate(pieces, axis=1)

    h = jnp.maximum(jnp.dot(r, tw0[...], preferred_element_type=f32) + tb0[...], 0.0)
    h = jnp.maximum(jnp.dot(h, tw1[...], preferred_element_type=f32) + tb1[...], 0.0)
    z = jnp.dot(h, tw2[...], preferred_element_type=f32) + tb2[...]
    out_ref[...] = 1.0 / (1.0 + jnp.exp(-z))


def _tc_call(dense_x, ly, weights, block_b):
    grid = (B // block_b,)
    full = lambda shape: pl.BlockSpec(shape, lambda i: (0,) * len(shape))
    in_specs = [
        pl.BlockSpec((block_b, 13), lambda i: (i, 0)),
        pl.BlockSpec((block_b, SLOTS, D), lambda i: (i, 0, 0)),
    ] + [full(w.shape) for w in weights]
    return pl.pallas_call(
        _tc_body,
        grid=grid,
        in_specs=in_specs,
        out_specs=pl.BlockSpec((block_b, 1), lambda i: (i, 0)),
        out_shape=jax.ShapeDtypeStruct((B, 1), jnp.float32),
    )(dense_x, ly, *weights)


def _arrange_top_w0(top_w0):
    w0t = top_w0.T
    segs = [w0t[:D]]
    off = D
    for i in range(1, NT + 1):
        segs.append(w0t[off:off + i])
        segs.append(jnp.zeros((SLOTS - i, w0t.shape[1]), jnp.float32))
        off += i
    return jnp.concatenate(segs, axis=0)


def kernel(dense_x, lS_o, lS_i, emb_tables,
           bot_w0, bot_b0, bot_w1, bot_b1, bot_w2, bot_b2,
           top_w0, top_b0, top_w1, top_b1, top_w2, top_b2):
    del lS_o
    table_flat = emb_tables.reshape(NT * V, D)
    ly = _sc_gather()(lS_i, table_flat)

    weights = [
        bot_w0.T, bot_b0[None, :],
        bot_w1.T, bot_b1[None, :],
        bot_w2.T, bot_b2[None, :],
        _arrange_top_w0(top_w0), top_b0[None, :],
        top_w1.T, top_b1[None, :],
        top_w2.T, top_b2[None, :],
    ]
    return _tc_call(dense_x, ly, weights, block_b=256)


# speedup vs baseline: 1.3861x; 1.3861x over previous
import functools

import jax
import jax.numpy as jnp
from jax import lax
from jax.experimental import pallas as pl
from jax.experimental.pallas import tpu as pltpu
from jax.experimental.pallas import tpu_sc as plsc

B = 4096
NT = 26
V = 100000
D = 64

NC = 2
NS = 16
NW = NC * NS
ROWS = NT * D
RPW = ROWS // NW
HALF = 51200
SLOTS = 32


def _sc_gather_kernel(idx_hbm, table_hbm, out_hbm, idx_v, buf, outv):
    wid = lax.axis_index("s") * NC + lax.axis_index("c")

    def do_row(k, carry):
        u = wid * RPW + k
        t = u // D
        d = u % D
        pltpu.sync_copy(idx_hbm.at[t, :], idx_v)
        for c0, w in ((0, HALF), (HALF, V - HALF)):
            pltpu.sync_copy(table_hbm.at[t, d, pl.ds(c0, w)],
                            buf.at[pl.ds(0, w)])

            def do_vec(j, carry2):
                sl = pl.ds(j * 16, 16)
                iv = idx_v[sl]
                m = (iv >= c0) & (iv < c0 + w)
                loc = jnp.where(m, iv - c0, 0)
                g = plsc.load_gather(buf, [loc], mask=m)
                outv[sl] = jnp.where(m, g, outv[sl])
                return carry2

            lax.fori_loop(0, B // 16, do_vec, 0)
        pltpu.sync_copy(outv, out_hbm.at[t, d, :])
        return carry

    lax.fori_loop(0, RPW, do_row, 0)


@functools.lru_cache(maxsize=None)
def _sc_gather():
    return pl.kernel(
        _sc_gather_kernel,
        mesh=plsc.VectorSubcoreMesh(core_axis_name="c", subcore_axis_name="s"),
        compiler_params=pltpu.CompilerParams(
            use_tc_tiling_on_sc=False, needs_layout_passes=False),
        out_type=jax.ShapeDtypeStruct((NT, D, B), jnp.float32),
        scratch_types=[
            pltpu.VMEM((B,), jnp.int32),
            pltpu.VMEM((HALF,), jnp.float32),
            pltpu.VMEM((B,), jnp.float32),
        ],
    )


def _tc_body(denseT_ref, lyT_ref, bw0, bb0, bw1, bb1, bw2, bb2,
             tw0a, tb0, tw1, tb1, tw2, tb2, outT_ref):
    f32 = jnp.float32
    h = jnp.maximum(jnp.dot(bw0[...], denseT_ref[...],
                            preferred_element_type=f32) + bb0[...], 0.0)
    h = jnp.maximum(jnp.dot(bw1[...], h,
                            preferred_element_type=f32) + bb1[...], 0.0)
    xT = jnp.maximum(jnp.dot(bw2[...], h,
                             preferred_element_type=f32) + bb2[...], 0.0)

    bb = xT.shape[1]
    tallT = jnp.concatenate(
        [xT[None], lyT_ref[...], jnp.zeros((SLOTS - 1 - NT, D, bb), f32)],
        axis=0)

    pieces = [xT]
    for i in range(1, NT + 1):
        zi = jnp.sum(tallT * tallT[i:i + 1], axis=1)
        pieces.append(zi)
    rT = jnp.concatenate(pieces, axis=0)

    h = jnp.maximum(jnp.dot(tw0a[...], rT,
                            preferred_element_type=f32) + tb0[...], 0.0)
    h = jnp.maximum(jnp.dot(tw1[...], h,
                            preferred_element_type=f32) + tb1[...], 0.0)
    z = jnp.sum(h * tw2[...], axis=0, keepdims=True) + tb2[...]
    outT_ref[...] = 1.0 / (1.0 + jnp.exp(-z))


def _tc_call(denseT, lyT, weights, block_b):
    grid = (B // block_b,)
    full = lambda shape: pl.BlockSpec(shape, lambda i: (0,) * len(shape))
    in_specs = [
        pl.BlockSpec((13, block_b), lambda i: (0, i)),
        pl.BlockSpec((NT, D, block_b), lambda i: (0, 0, i)),
    ] + [full(w.shape) for w in weights]
    return pl.pallas_call(
        _tc_body,
        grid=grid,
        in_specs=in_specs,
        out_specs=pl.BlockSpec((1, block_b), lambda i: (0, i)),
        out_shape=jax.ShapeDtypeStruct((1, B), jnp.float32),
    )(denseT, lyT, *weights)


def _arrange_top_w0(top_w0):
    segs = [top_w0[:, :D]]
    off = D
    for i in range(1, NT + 1):
        segs.append(top_w0[:, off:off + i])
        segs.append(jnp.zeros((top_w0.shape[0], SLOTS - i), jnp.float32))
        off += i
    return jnp.concatenate(segs, axis=1)


def kernel(dense_x, lS_o, lS_i, emb_tables,
           bot_w0, bot_b0, bot_w1, bot_b1, bot_w2, bot_b2,
           top_w0, top_b0, top_w1, top_b1, top_w2, top_b2):
    del lS_o
    table_t = jnp.transpose(emb_tables, (0, 2, 1))
    lyT = _sc_gather()(lS_i, table_t)

    weights = [
        bot_w0, bot_b0[:, None],
        bot_w1, bot_b1[:, None],
        bot_w2, bot_b2[:, None],
        _arrange_top_w0(top_w0), top_b0[:, None],
        top_w1, top_b1[:, None],
        top_w2[0][:, None], top_b2[:, None],
    ]
    outT = _tc_call(dense_x.T, lyT, weights, block_b=256)
    return outT.reshape(B, 1)


# speedup vs baseline: 4.8065x; 3.4678x over previous
import functools

import jax
import jax.numpy as jnp
from jax import lax
from jax.experimental import pallas as pl
from jax.experimental.pallas import tpu as pltpu
from jax.experimental.pallas import tpu_sc as plsc

B = 4096
NT = 26
V = 100000
D = 64

NC = 2
NS = 16
NW = NC * NS
ROWS = NT * D
RPW = ROWS // NW
MAIN = 99968
TAILW = 128
TAIL0 = V - TAILW
SLOTS = 32


NBUF = 4
C0W = 25088
_cs = []
_o = 0
while _o < MAIN:
    _w = min(C0W, MAIN - _o)
    _cs.append((_o, _w))
    _o += _w
CHUNKS = tuple(_cs)


def _sc_gather_kernel(idx_hbm, table_hbm, tail_hbm, out_hbm,
                      idx_v, buf0, buf1, buf2, buf3, tbuf, outv,
                      sem0, sem1, sem2, sem3):
    wid = lax.axis_index("s") * NC + lax.axis_index("c")
    base = wid * RPW
    bufs = (buf0, buf1, buf2, buf3)
    sems = (sem0, sem1, sem2, sem3)
    nch = len(CHUNKS)
    nsteps = RPW * nch

    def td(k):
        u = base + k
        return u // D, u % D

    def start(c):
        t, d = td(c // nch)
        c0, w = CHUNKS[c % nch]
        return pltpu.async_copy(table_hbm.at[t, d, pl.ds(c0, w)],
                                bufs[c % NBUF].at[pl.ds(0, w)], sems[c % NBUF])

    for c in range(NBUF - 1):
        start(c)

    def do_row(k, carry):
        t, d = td(k)
        pltpu.sync_copy(idx_hbm.at[t, :], idx_v)
        pltpu.sync_copy(tail_hbm.at[t, d, :], tbuf)
        for ci, (c0, w) in enumerate(CHUNKS):
            ca = ci + NBUF - 1
            k_pre = k + ca // nch

            @pl.when(k_pre < RPW)
            def _():
                t2, d2 = td(k_pre)
                c0p, wp = CHUNKS[ca % nch]
                pltpu.async_copy(table_hbm.at[t2, d2, pl.ds(c0p, wp)],
                                 bufs[ca % NBUF].at[pl.ds(0, wp)],
                                 sems[ca % NBUF])

            buf = bufs[ci]
            pltpu.make_async_copy(table_hbm.at[0, 0, pl.ds(c0, w)],
                                  buf.at[pl.ds(0, w)], sems[ci]).wait()
            if ci == 0:
                def do_vec0(j, carry2):
                    sl = pl.ds(j * 16, 16)
                    iv = idx_v[sl]
                    m0 = iv < C0W
                    mt = iv >= MAIN
                    g = plsc.load_gather(buf, [jnp.where(m0, iv, 0)], mask=m0)
                    gt = plsc.load_gather(
                        tbuf, [jnp.where(mt, iv - TAIL0, 0)], mask=mt)
                    outv[sl] = jnp.where(m0, g, jnp.where(mt, gt, g))
                    return carry2
                lax.fori_loop(0, B // 16, do_vec0, 0)
            else:
                def do_vec1(j, carry2):
                    sl = pl.ds(j * 16, 16)
                    iv = idx_v[sl]
                    m1 = (iv >= c0) & (iv < c0 + w)
                    g = plsc.load_gather(
                        buf, [jnp.where(m1, iv - c0, 0)], mask=m1)
                    outv[sl] = jnp.where(m1, g, outv[sl])
                    return carry2
                lax.fori_loop(0, B // 16, do_vec1, 0)
        pltpu.sync_copy(outv, out_hbm.at[t, d, :])
        return carry

    lax.fori_loop(0, RPW, do_row, 0)


@functools.lru_cache(maxsize=None)
def _sc_gather():
    return pl.kernel(
        _sc_gather_kernel,
        mesh=plsc.VectorSubcoreMesh(core_axis_name="c", subcore_axis_name="s"),
        compiler_params=pltpu.CompilerParams(needs_layout_passes=False),
        out_type=jax.ShapeDtypeStruct((NT, D, B), jnp.float32),
        scratch_types=[
            pltpu.VMEM((B,), jnp.int32),
            pltpu.VMEM((C0W,), jnp.float32),
            pltpu.VMEM((C0W,), jnp.float32),
            pltpu.VMEM((C0W,), jnp.float32),
            pltpu.VMEM((C0W,), jnp.float32),
            pltpu.VMEM((TAILW,), jnp.float32),
            pltpu.VMEM((B,), jnp.float32),
            pltpu.SemaphoreType.DMA,
            pltpu.SemaphoreType.DMA,
            pltpu.SemaphoreType.DMA,
            pltpu.SemaphoreType.DMA,
        ],
    )


def _tc_body(denseT_ref, lyT_ref, bw0, bb0, bw1, bb1, bw2, bb2,
             tw0a, tb0, tw1, tb1, tw2, tb2, outT_ref):
    f32 = jnp.float32
    h = jnp.maximum(jnp.dot(bw0[...], denseT_ref[...],
                            preferred_element_type=f32) + bb0[...], 0.0)
    h = jnp.maximum(jnp.dot(bw1[...], h,
                            preferred_element_type=f32) + bb1[...], 0.0)
    xT = jnp.maximum(jnp.dot(bw2[...], h,
                             preferred_element_type=f32) + bb2[...], 0.0)

    bb = xT.shape[1]
    tallT = jnp.concatenate(
        [xT[None], lyT_ref[...], jnp.zeros((SLOTS - 1 - NT, D, bb), f32)],
        axis=0)

    pieces = [xT]
    for i in range(1, NT + 1):
        zi = jnp.sum(tallT * tallT[i:i + 1], axis=1)
        pieces.append(zi)
    rT = jnp.concatenate(pieces, axis=0)

    h = jnp.maximum(jnp.dot(tw0a[...], rT,
                            preferred_element_type=f32) + tb0[...], 0.0)
    h = jnp.maximum(jnp.dot(tw1[...], h,
                            preferred_element_type=f32) + tb1[...], 0.0)
    z = jnp.sum(h * tw2[...], axis=0, keepdims=True) + tb2[...]
    outT_ref[...] = 1.0 / (1.0 + jnp.exp(-z))


def _tc_call(denseT, lyT, weights, block_b):
    grid = (B // block_b,)
    full = lambda shape: pl.BlockSpec(shape, lambda i: (0,) * len(shape))
    in_specs = [
        pl.BlockSpec((13, block_b), lambda i: (0, i)),
        pl.BlockSpec((NT, D, block_b), lambda i: (0, 0, i)),
    ] + [full(w.shape) for w in weights]
    return pl.pallas_call(
        _tc_body,
        grid=grid,
        in_specs=in_specs,
        out_specs=pl.BlockSpec((1, block_b), lambda i: (0, i)),
        out_shape=jax.ShapeDtypeStruct((1, B), jnp.float32),
    )(denseT, lyT, *weights)


def _arrange_top_w0(top_w0):
    segs = [top_w0[:, :D]]
    off = D
    for i in range(1, NT + 1):
        segs.append(top_w0[:, off:off + i])
        segs.append(jnp.zeros((top_w0.shape[0], SLOTS - i), jnp.float32))
        off += i
    return jnp.concatenate(segs, axis=1)


def kernel(dense_x, lS_o, lS_i, emb_tables,
           bot_w0, bot_b0, bot_w1, bot_b1, bot_w2, bot_b2,
           top_w0, top_b0, top_w1, top_b1, top_w2, top_b2):
    del lS_o
    table_t = jnp.transpose(emb_tables, (0, 2, 1))
    tail_t = lax.slice(table_t, (0, 0, TAIL0), (NT, D, V))
    lyT = _sc_gather()(lS_i, table_t, tail_t)

    weights = [
        bot_w0, bot_b0[:, None],
        bot_w1, bot_b1[:, None],
        bot_w2, bot_b2[:, None],
        _arrange_top_w0(top_w0), top_b0[:, None],
        top_w1, top_b1[:, None],
        top_w2[0][:, None], top_b2[:, None],
    ]
    outT = _tc_call(dense_x.T, lyT, weights, block_b=256)
    return outT.reshape(B, 1)


# speedup vs baseline: 4.8901x; 1.0174x over previous
import functools

import jax
import jax.numpy as jnp
from jax import lax
from jax.experimental import pallas as pl
from jax.experimental.pallas import tpu as pltpu
from jax.experimental.pallas import tpu_sc as plsc

B = 4096
NT = 26
V = 100000
D = 64

NC = 2
NS = 16
NW = NC * NS
ROWS = NT * D
RPW = ROWS // NW
MAIN = 99968
TAILW = 128
TAIL0 = V - TAILW
SLOTS = 32


NBUF = 4
C0W = 25088
_cs = []
_o = 0
while _o < MAIN:
    _w = min(C0W, MAIN - _o)
    _cs.append((_o, _w))
    _o += _w
CHUNKS = tuple(_cs)


def _sc_gather_kernel(idx_hbm, table_hbm, tail_hbm, out_hbm,
                      idx_v, buf0, buf1, buf2, buf3, tbuf, outv,
                      sem0, sem1, sem2, sem3):
    wid = lax.axis_index("s") * NC + lax.axis_index("c")
    base = wid * RPW
    bufs = (buf0, buf1, buf2, buf3)
    sems = (sem0, sem1, sem2, sem3)
    nch = len(CHUNKS)
    nsteps = RPW * nch

    def td(k):
        u = base + k
        return u // D, u % D

    def start(c):
        t, d = td(c // nch)
        c0, w = CHUNKS[c % nch]
        return pltpu.async_copy(table_hbm.at[t, d, pl.ds(c0, w)],
                                bufs[c % NBUF].at[pl.ds(0, w)], sems[c % NBUF])

    for c in range(NBUF - 1):
        start(c)

    def do_row(k, carry):
        t, d = td(k)
        pltpu.sync_copy(idx_hbm.at[t, :], idx_v)
        pltpu.sync_copy(tail_hbm.at[t, d, :], tbuf)
        for ci, (c0, w) in enumerate(CHUNKS):
            ca = ci + NBUF - 1
            k_pre = k + ca // nch

            @pl.when(k_pre < RPW)
            def _():
                t2, d2 = td(k_pre)
                c0p, wp = CHUNKS[ca % nch]
                pltpu.async_copy(table_hbm.at[t2, d2, pl.ds(c0p, wp)],
                                 bufs[ca % NBUF].at[pl.ds(0, wp)],
                                 sems[ca % NBUF])

            buf = bufs[ci]
            pltpu.make_async_copy(table_hbm.at[0, 0, pl.ds(c0, w)],
                                  buf.at[pl.ds(0, w)], sems[ci]).wait()
            if ci == 0:
                def do_vec0(j, carry2):
                    sl = pl.ds(j * 16, 16)
                    iv = idx_v[sl]
                    m0 = iv < C0W
                    mt = iv >= MAIN
                    g = plsc.load_gather(buf, [jnp.where(m0, iv, 0)], mask=m0)
                    gt = plsc.load_gather(
                        tbuf, [jnp.where(mt, iv - TAIL0, 0)], mask=mt)
                    outv[sl] = jnp.where(m0, g, jnp.where(mt, gt, g))
                    return carry2
                lax.fori_loop(0, B // 16, do_vec0, 0)
            else:
                def do_vec1(j, carry2):
                    sl = pl.ds(j * 16, 16)
                    iv = idx_v[sl]
                    m1 = (iv >= c0) & (iv < c0 + w)
                    g = plsc.load_gather(
                        buf, [jnp.where(m1, iv - c0, 0)], mask=m1)
                    outv[sl] = jnp.where(m1, g, outv[sl])
                    return carry2
                lax.fori_loop(0, B // 16, do_vec1, 0)
        pltpu.sync_copy(outv, out_hbm.at[t, d, :])
        return carry

    lax.fori_loop(0, RPW, do_row, 0)


@functools.lru_cache(maxsize=None)
def _sc_gather():
    return pl.kernel(
        _sc_gather_kernel,
        mesh=plsc.VectorSubcoreMesh(core_axis_name="c", subcore_axis_name="s"),
        compiler_params=pltpu.CompilerParams(needs_layout_passes=False),
        out_type=jax.ShapeDtypeStruct((NT, D, B), jnp.float32),
        scratch_types=[
            pltpu.VMEM((B,), jnp.int32),
            pltpu.VMEM((C0W,), jnp.float32),
            pltpu.VMEM((C0W,), jnp.float32),
            pltpu.VMEM((C0W,), jnp.float32),
            pltpu.VMEM((C0W,), jnp.float32),
            pltpu.VMEM((TAILW,), jnp.float32),
            pltpu.VMEM((B,), jnp.float32),
            pltpu.SemaphoreType.DMA,
            pltpu.SemaphoreType.DMA,
            pltpu.SemaphoreType.DMA,
            pltpu.SemaphoreType.DMA,
        ],
    )


def _tc_body(denseT_ref, lyT_ref, bw0, bb0, bw1, bb1, bw2, bb2,
             tw0a, tb0, tw1, tb1, tw2, tb2, outT_ref):
    f32 = jnp.float32
    h = jnp.maximum(jnp.dot(bw0[...], denseT_ref[...],
                            preferred_element_type=f32) + bb0[...], 0.0)
    h = jnp.maximum(jnp.dot(bw1[...], h,
                            preferred_element_type=f32) + bb1[...], 0.0)
    xT = jnp.maximum(jnp.dot(bw2[...], h,
                             preferred_element_type=f32) + bb2[...], 0.0)

    bb = xT.shape[1]
    tallT = jnp.concatenate(
        [xT[None], lyT_ref[...], jnp.zeros((SLOTS - 1 - NT, D, bb), f32)],
        axis=0)

    pieces = [xT]
    for i in range(1, NT + 1):
        ni = -(-(i + 1) // 8) * 8
        zi = jnp.sum(tallT[:ni] * tallT[i:i + 1], axis=1)
        pieces.append(zi)
    rT = jnp.concatenate(pieces, axis=0)

    h = jnp.maximum(jnp.dot(tw0a[...], rT,
                            preferred_element_type=f32) + tb0[...], 0.0)
    h = jnp.maximum(jnp.dot(tw1[...], h,
                            preferred_element_type=f32) + tb1[...], 0.0)
    z = jnp.sum(h * tw2[...], axis=0, keepdims=True) + tb2[...]
    outT_ref[...] = 1.0 / (1.0 + jnp.exp(-z))


def _tc_call(denseT, lyT, weights, block_b):
    grid = (B // block_b,)
    full = lambda shape: pl.BlockSpec(shape, lambda i: (0,) * len(shape))
    in_specs = [
        pl.BlockSpec((13, block_b), lambda i: (0, i)),
        pl.BlockSpec((NT, D, block_b), lambda i: (0, 0, i)),
    ] + [full(w.shape) for w in weights]
    return pl.pallas_call(
        _tc_body,
        grid=grid,
        in_specs=in_specs,
        out_specs=pl.BlockSpec((1, block_b), lambda i: (0, i)),
        out_shape=jax.ShapeDtypeStruct((1, B), jnp.float32),
    )(denseT, lyT, *weights)


def _arrange_top_w0(top_w0):
    segs = [top_w0[:, :D]]
    off = D
    for i in range(1, NT + 1):
        ni = -(-(i + 1) // 8) * 8
        segs.append(top_w0[:, off:off + i])
        segs.append(jnp.zeros((top_w0.shape[0], ni - i), jnp.float32))
        off += i
    return jnp.concatenate(segs, axis=1)


def kernel(dense_x, lS_o, lS_i, emb_tables,
           bot_w0, bot_b0, bot_w1, bot_b1, bot_w2, bot_b2,
           top_w0, top_b0, top_w1, top_b1, top_w2, top_b2):
    del lS_o
    table_t = jnp.transpose(emb_tables, (0, 2, 1))
    tail_t = lax.slice(table_t, (0, 0, TAIL0), (NT, D, V))
    lyT = _sc_gather()(lS_i, table_t, tail_t)

    weights = [
        bot_w0, bot_b0[:, None],
        bot_w1, bot_b1[:, None],
        bot_w2, bot_b2[:, None],
        _arrange_top_w0(top_w0), top_b0[:, None],
        top_w1, top_b1[:, None],
        top_w2[0][:, None], top_b2[:, None],
    ]
    outT = _tc_call(dense_x.T, lyT, weights, block_b=256)
    return outT.reshape(B, 1)
